# baseline (speedup 1.0000x reference)
import functools

import jax
import jax.numpy as jnp
from jax import lax
from jax.experimental import pallas as pl
from jax.experimental.pallas import tpu as pltpu
from jax.experimental.pallas import tpu_sc as plsc

INPUT_DIM = 1000000
MODEL_DIM = 64
SEQ_LEN = 200
BATCH = 4096
LANES = 16
D_VECS = MODEL_DIM // LANES
SCALE = float(MODEL_DIM) ** 0.5

_info = plsc.get_sparse_core_info()
NC, NS = _info.num_cores, _info.num_subcores
NW = NC * NS

_mesh = plsc.VectorSubcoreMesh(core_axis_name="c", subcore_axis_name="s")

BLK = 128
NBLK_FULL = INPUT_DIM // BLK
TAIL = INPUT_DIM - NBLK_FULL * BLK
BLK_W = NBLK_FULL // NW
NBLK_EXTRA = NBLK_FULL - BLK_W * NW


@functools.partial(
    pl.kernel,
    out_type=jax.ShapeDtypeStruct((INPUT_DIM, 2 * MODEL_DIM), jnp.float32),
    mesh=_mesh,
    compiler_params=pltpu.CompilerParams(needs_layout_passes=False),
    scratch_types=[
        pltpu.VMEM((2, MODEL_DIM, BLK), jnp.float32),
        pltpu.VMEM((2, BLK, 2 * MODEL_DIM), jnp.float32),
        pltpu.SemaphoreType.DMA,
        pltpu.SemaphoreType.DMA,
        pltpu.SemaphoreType.DMA,
        pltpu.SemaphoreType.DMA,
    ],
)
def _sc_stage_table(tt_hbm, tail_hbm, out_hbm, bin_v, bout_v,
                    i_sem0, i_sem1, o_sem0, o_sem1):
    wid = lax.axis_index("s") * NC + lax.axis_index("c")
    base = wid * BLK_W
    i_sems = (i_sem0, i_sem1)
    o_sems = (o_sem0, o_sem1)
    lane = lax.iota(jnp.int32, LANES)

    def in_copy(j, k):
        return pltpu.make_async_copy(
            tt_hbm.at[:, pl.ds(j * BLK, BLK)], bin_v.at[k], i_sems[k])

    def out_copy(j, k):
        return pltpu.make_async_copy(
            bout_v.at[k], out_hbm.at[pl.ds(j * BLK, BLK)], o_sems[k])

    def transpose_block(k):
        ksplat = jnp.full((LANES,), k, jnp.int32)

        def row_body(r, carry):
            rsplat = jnp.full((LANES,), r, jnp.int32)
            for v in range(D_VECS):
                vec = plsc.load_gather(
                    bin_v, [ksplat, lane + (v * LANES), rsplat]) * SCALE
                bout_v[k, r, pl.ds(v * LANES, LANES)] = vec
                bout_v[k, r, pl.ds(MODEL_DIM + v * LANES, LANES)] = vec
            return carry

        lax.fori_loop(0, BLK, row_body, 0)

    in_copy(base, 0).start()

    def blk_body(i, carry):
        j = base + i
        k = lax.rem(i, 2)
        kn = 1 - k
        @pl.when(k == 0)
        def _():
            in_copy(j, 0).wait()
            with jax.named_scope("prefetch"):
                @pl.when(i + 1 < BLK_W)
                def _():
                    in_copy(j + 1, 1).start()
            @pl.when(i >= 2)
            def _():
                out_copy(j, 0).wait()
            transpose_block(0)
            out_copy(j, 0).start()

        @pl.when(k == 1)
        def _():
            in_copy(j, 1).wait()
            @pl.when(i + 1 < BLK_W)
            def _():
                in_copy(j + 1, 0).start()
            @pl.when(i >= 2)
            def _():
                out_copy(j, 1).wait()
            transpose_block(1)
            out_copy(j, 1).start()

        return carry

    lax.fori_loop(0, BLK_W, blk_body, 0)
    out_copy(base + BLK_W - 2, 0).wait()
    out_copy(base + BLK_W - 1, 1).wait()

    @pl.when(wid < NBLK_EXTRA)
    def _():
        j = NW * BLK_W + wid
        in_copy(j, 0).start()
        in_copy(j, 0).wait()
        transpose_block(0)
        out_copy(j, 0).start()
        out_copy(j, 0).wait()

    @pl.when(wid == NW - 1)
    def _():
        c0 = NBLK_FULL * BLK
        tcp = pltpu.make_async_copy(
            tail_hbm, bout_v.at[1, pl.ds(0, TAIL)], i_sems[1])
        tcp.start()
        tcp.wait()
        ocp = pltpu.make_async_copy(
            bout_v.at[1, pl.ds(0, TAIL)],
            out_hbm.at[pl.ds(c0, TAIL)], o_sems[1])
        ocp.start()
        ocp.wait()


XROWS_W = BATCH // NW
CHUNK_X = 2
NCHUNK = XROWS_W // CHUNK_X
NBUF = 2
G_SPLITS = ((0, 128), (128, 72))


@functools.partial(
    pl.kernel,
    out_type=jax.ShapeDtypeStruct((BATCH, SEQ_LEN, 2 * MODEL_DIM),
                                   jnp.float32),
    mesh=_mesh,
    scratch_types=[
        pltpu.VMEM((NBUF, CHUNK_X, SEQ_LEN), jnp.int32),
        pltpu.VMEM((NBUF, CHUNK_X, SEQ_LEN, 2 * MODEL_DIM), jnp.float32),
        pltpu.VMEM((SEQ_LEN, MODEL_DIM), jnp.float32),
        pltpu.SemaphoreType.DMA,
        pltpu.SemaphoreType.DMA,
        pltpu.SemaphoreType.DMA,
        pltpu.SemaphoreType.DMA,
    ],
)
def _sc_gather(x_hbm, table_hbm, pos_hbm, out_hbm, idx_v, rows_v, pos_v,
               g_sem0, g_sem1, o_sem0, o_sem1):
    wid = lax.axis_index("s") * NC + lax.axis_index("c")
    x_base = wid * XROWS_W
    g_sems = (g_sem0, g_sem1)
    o_sems = (o_sem0, o_sem1)

    pltpu.sync_copy(pos_hbm.at[0], pos_v)

    gathers = [None] * NBUF
    out_cps = [None] * NBUF

    def start_chunk(c):
        k = c % NBUF
        b = x_base + c * CHUNK_X
        pltpu.sync_copy(x_hbm.at[pl.ds(b, CHUNK_X), :], idx_v.at[k])
        cps = []
        for r in range(CHUNK_X):
            for (off, ln) in G_SPLITS:
                cps.append(pltpu.async_copy(
                    table_hbm.at[idx_v.at[k, r, pl.ds(off, ln)]],
                    rows_v.at[k, r, pl.ds(off, ln)],
                    g_sems[k],
                ))
        gathers[k] = cps

    def finish_chunk(c):
        k = c % NBUF
        for cp in gathers[k]:
            cp.wait()

        def body(s, carry):
            for d in range(D_VECS):
                pv = pos_v[s, pl.ds(d * LANES, LANES)]
                for r in range(CHUNK_X):
                    v = rows_v[k, r, s, pl.ds(d * LANES, LANES)]
                    rows_v[k, r, s, pl.ds(d * LANES, LANES)] = v + pv
            return carry

        lax.fori_loop(0, SEQ_LEN, body, 0)
        b = x_base + c * CHUNK_X
        out_cps[k] = pltpu.async_copy(
            rows_v.at[k], out_hbm.at[pl.ds(b, CHUNK_X)], o_sems[k])

    for c in range(NCHUNK):
        k = c % NBUF
        if out_cps[k] is not None:
            out_cps[k].wait()
            out_cps[k] = None
        start_chunk(c)
        if c >= 1:
            finish_chunk(c - 1)
    finish_chunk(NCHUNK - 1)
    for k in range(NBUF):
        if out_cps[k] is not None:
            out_cps[k].wait()


def kernel(x, embedding, positional_encoding):
    tail128 = jnp.tile(embedding[NBLK_FULL * BLK:] * SCALE, (1, 2))
    table128 = _sc_stage_table(embedding.T, tail128)
    wide = _sc_gather(x, table128, positional_encoding)
    return wide[:, :, :MODEL_DIM]


# speedup vs baseline: 2.1782x; 2.1782x over previous
import functools

import jax
import jax.numpy as jnp
from jax import lax
from jax.experimental import pallas as pl
from jax.experimental.pallas import tpu as pltpu
from jax.experimental.pallas import tpu_sc as plsc

INPUT_DIM = 1000000
MODEL_DIM = 64
SEQ_LEN = 200
BATCH = 4096
LANES = 16
D_VECS = MODEL_DIM // LANES
SCALE = float(MODEL_DIM) ** 0.5

_info = plsc.get_sparse_core_info()
NC, NS = _info.num_cores, _info.num_subcores
NW = NC * NS
XROWS_W = BATCH // NW
CHUNK_X = 2
NCHUNK = XROWS_W // CHUNK_X
NBUF = 2
G_SPLITS = ((0, 128), (128, 72))

_mesh = plsc.VectorSubcoreMesh(core_axis_name="c", subcore_axis_name="s")

_TR_COLS = 2048


def _transpose_body(tin_ref, tout_ref):
    t = tin_ref[...]
    eye = jnp.eye(MODEL_DIM, dtype=jnp.float32) * SCALE
    tt = jax.lax.dot_general(t, eye, (((0,), (0,)), ((), ())),
                             preferred_element_type=jnp.float32)
    tout_ref[...] = jnp.concatenate([tt, tt], axis=1)


def _stage_table(embedding):
    return pl.pallas_call(
        _transpose_body,
        grid=(pl.cdiv(INPUT_DIM, _TR_COLS),),
        in_specs=[pl.BlockSpec((MODEL_DIM, _TR_COLS), lambda i: (0, i))],
        out_specs=pl.BlockSpec((_TR_COLS, 2 * MODEL_DIM), lambda i: (i, 0)),
        out_shape=jax.ShapeDtypeStruct((INPUT_DIM, 2 * MODEL_DIM),
                                       jnp.float32),
    )(embedding.T)


@functools.partial(
    pl.kernel,
    out_type=jax.ShapeDtypeStruct((BATCH, SEQ_LEN, 2 * MODEL_DIM),
                                   jnp.float32),
    mesh=_mesh,
    scratch_types=[
        pltpu.VMEM((NBUF, CHUNK_X, SEQ_LEN), jnp.int32),
        pltpu.VMEM((NBUF, CHUNK_X, SEQ_LEN, 2 * MODEL_DIM), jnp.float32),
        pltpu.VMEM((SEQ_LEN, MODEL_DIM), jnp.float32),
        pltpu.SemaphoreType.DMA,
        pltpu.SemaphoreType.DMA,
        pltpu.SemaphoreType.DMA,
        pltpu.SemaphoreType.DMA,
    ],
)
def _sc_gather(x_hbm, table_hbm, pos_hbm, out_hbm, idx_v, rows_v, pos_v,
               g_sem0, g_sem1, o_sem0, o_sem1):
    wid = lax.axis_index("s") * NC + lax.axis_index("c")
    x_base = wid * XROWS_W
    g_sems = (g_sem0, g_sem1)
    o_sems = (o_sem0, o_sem1)

    pltpu.sync_copy(pos_hbm.at[0], pos_v)

    gathers = [None] * NBUF
    out_cps = [None] * NBUF

    def start_chunk(c):
        k = c % NBUF
        b = x_base + c * CHUNK_X
        pltpu.sync_copy(x_hbm.at[pl.ds(b, CHUNK_X), :], idx_v.at[k])
        cps = []
        for r in range(CHUNK_X):
            for (off, ln) in G_SPLITS:
                cps.append(pltpu.async_copy(
                    table_hbm.at[idx_v.at[k, r, pl.ds(off, ln)]],
                    rows_v.at[k, r, pl.ds(off, ln)],
                    g_sems[k],
                ))
        gathers[k] = cps

    def finish_chunk(c):
        k = c % NBUF
        for cp in gathers[k]:
            cp.wait()

        def body(s, carry):
            for d in range(D_VECS):
                pv = pos_v[s, pl.ds(d * LANES, LANES)]
                for r in range(CHUNK_X):
                    v = rows_v[k, r, s, pl.ds(d * LANES, LANES)]
                    rows_v[k, r, s, pl.ds(d * LANES, LANES)] = v + pv
            return carry

        lax.fori_loop(0, SEQ_LEN, body, 0)
        b = x_base + c * CHUNK_X
        out_cps[k] = pltpu.async_copy(
            rows_v.at[k], out_hbm.at[pl.ds(b, CHUNK_X)], o_sems[k])

    for c in range(NCHUNK):
        k = c % NBUF
        if out_cps[k] is not None:
            out_cps[k].wait()
            out_cps[k] = None
        start_chunk(c)
        if c >= 1:
            finish_chunk(c - 1)
    finish_chunk(NCHUNK - 1)
    for k in range(NBUF):
        if out_cps[k] is not None:
            out_cps[k].wait()


def kernel(x, embedding, positional_encoding):
    table128 = _stage_table(embedding)
    wide = _sc_gather(x, table128, positional_encoding)
    return wide[:, :, :MODEL_DIM]


# speedup vs baseline: 2.1813x; 1.0014x over previous
import functools

import jax
import jax.numpy as jnp
from jax import lax
from jax.experimental import pallas as pl
from jax.experimental.pallas import tpu as pltpu
from jax.experimental.pallas import tpu_sc as plsc

INPUT_DIM = 1000000
MODEL_DIM = 64
SEQ_LEN = 200
BATCH = 4096
LANES = 16
D_VECS = MODEL_DIM // LANES
SCALE = float(MODEL_DIM) ** 0.5

_info = plsc.get_sparse_core_info()
NC, NS = _info.num_cores, _info.num_subcores
NW = NC * NS
XROWS_W = BATCH // NW
CHUNK_X = 2
NCHUNK = XROWS_W // CHUNK_X
NBUF = 2
G_SPLITS = ((0, 128), (128, 72))

_mesh = plsc.VectorSubcoreMesh(core_axis_name="c", subcore_axis_name="s")

_TR_COLS = 2048


def _transpose_body(tin_ref, tout_ref):
    t = tin_ref[...]
    tt = jnp.swapaxes(t, 0, 1) * SCALE
    tout_ref[...] = jnp.concatenate([tt, tt], axis=1)


def _stage_table(embedding):
    return pl.pallas_call(
        _transpose_body,
        grid=(pl.cdiv(INPUT_DIM, _TR_COLS),),
        in_specs=[pl.BlockSpec((MODEL_DIM, _TR_COLS), lambda i: (0, i))],
        out_specs=pl.BlockSpec((_TR_COLS, 2 * MODEL_DIM), lambda i: (i, 0)),
        out_shape=jax.ShapeDtypeStruct((INPUT_DIM, 2 * MODEL_DIM),
                                       jnp.float32),
    )(embedding.T)


@functools.partial(
    pl.kernel,
    out_type=jax.ShapeDtypeStruct((BATCH, SEQ_LEN, 2 * MODEL_DIM),
                                   jnp.float32),
    mesh=_mesh,
    scratch_types=[
        pltpu.VMEM((NBUF, CHUNK_X, SEQ_LEN), jnp.int32),
        pltpu.VMEM((NBUF, CHUNK_X, SEQ_LEN, 2 * MODEL_DIM), jnp.float32),
        pltpu.VMEM((SEQ_LEN, MODEL_DIM), jnp.float32),
        pltpu.SemaphoreType.DMA,
        pltpu.SemaphoreType.DMA,
        pltpu.SemaphoreType.DMA,
        pltpu.SemaphoreType.DMA,
    ],
)
def _sc_gather(x_hbm, table_hbm, pos_hbm, out_hbm, idx_v, rows_v, pos_v,
               g_sem0, g_sem1, o_sem0, o_sem1):
    wid = lax.axis_index("s") * NC + lax.axis_index("c")
    x_base = wid * XROWS_W
    g_sems = (g_sem0, g_sem1)
    o_sems = (o_sem0, o_sem1)

    pltpu.sync_copy(pos_hbm.at[0], pos_v)

    gathers = [None] * NBUF
    out_cps = [None] * NBUF

    def start_chunk(c):
        k = c % NBUF
        b = x_base + c * CHUNK_X
        pltpu.sync_copy(x_hbm.at[pl.ds(b, CHUNK_X), :], idx_v.at[k])
        cps = []
        for r in range(CHUNK_X):
            for (off, ln) in G_SPLITS:
                cps.append(pltpu.async_copy(
                    table_hbm.at[idx_v.at[k, r, pl.ds(off, ln)]],
                    rows_v.at[k, r, pl.ds(off, ln)],
                    g_sems[k],
                ))
        gathers[k] = cps

    def finish_chunk(c):
        k = c % NBUF
        for cp in gathers[k]:
            cp.wait()

        def body(s, carry):
            for d in range(D_VECS):
                pv = pos_v[s, pl.ds(d * LANES, LANES)]
                for r in range(CHUNK_X):
                    v = rows_v[k, r, s, pl.ds(d * LANES, LANES)]
                    rows_v[k, r, s, pl.ds(d * LANES, LANES)] = v + pv
            return carry

        lax.fori_loop(0, SEQ_LEN, body, 0)
        b = x_base + c * CHUNK_X
        out_cps[k] = pltpu.async_copy(
            rows_v.at[k], out_hbm.at[pl.ds(b, CHUNK_X)], o_sems[k])

    for c in range(NCHUNK):
        k = c % NBUF
        if out_cps[k] is not None:
            out_cps[k].wait()
            out_cps[k] = None
        start_chunk(c)
        if c >= 1:
            finish_chunk(c - 1)
    finish_chunk(NCHUNK - 1)
    for k in range(NBUF):
        if out_cps[k] is not None:
            out_cps[k].wait()


def kernel(x, embedding, positional_encoding):
    table128 = _stage_table(embedding)
    wide = _sc_gather(x, table128, positional_encoding)
    return wide[:, :, :MODEL_DIM]
